# baseline (speedup 1.0000x reference)
import functools

import jax
import jax.numpy as jnp
from jax import lax
from jax.experimental import pallas as pl
from jax.experimental.pallas import tpu as pltpu
from jax.experimental.pallas import tpu_sc as plsc

N = 10000
E = 320000
G = 16
W = 128
NC = 2
NS = 16
NPAD = 10112
ZROWS = NPAD // NS
K = 128
EP = 327680
DROW = 10016

_mesh = plsc.VectorSubcoreMesh(core_axis_name="c", subcore_axis_name="s")


WD = 16


def _make_deg_kernel():
    ep_tile = EP // (NC * NS)
    n = ep_tile // K
    nb = 2

    @functools.partial(
        pl.kernel,
        out_type=jax.ShapeDtypeStruct((NC * NPAD, WD), jnp.float32),
        mesh=_mesh,
        scratch_types=[
            pltpu.VMEM_SHARED((NPAD, WD), jnp.float32),
            pltpu.VMEM((nb, K), jnp.int32),
            pltpu.VMEM((K, WD), jnp.float32),
            pltpu.SemaphoreType.DMA((nb,)),
            pltpu.SemaphoreType.DMA((nb,)),
        ],
    )
    def kdeg(dst_hbm, zin_hbm, ones_hbm, out_hbm, acc, idx_d, ones_v,
             sem_i, sem_s):
        c = lax.axis_index("c")
        s = lax.axis_index("s")
        wid = s * NC + c
        pltpu.sync_copy(zin_hbm, acc.at[pl.ds(s * ZROWS, ZROWS)])
        pltpu.sync_copy(ones_hbm, ones_v)
        plsc.subcore_barrier()

        def stages(t, carry):
            @pl.when(t < n)
            def _a():
                b = lax.rem(t, nb)

                @pl.when(t >= nb)
                def _wait_free():
                    pltpu.make_async_copy(
                        ones_v, acc.at[idx_d.at[b]], sem_s.at[b]).wait()

                base = wid * ep_tile + t * K
                pltpu.async_copy(dst_hbm.at[pl.ds(base, K)],
                                 idx_d.at[b], sem_i.at[b])

            @pl.when(t >= 1)
            def _c():
                b = lax.rem(t - 1, nb)
                pltpu.make_async_copy(dst_hbm.at[pl.ds(0, K)],
                                      idx_d.at[b], sem_i.at[b]).wait()
                pltpu.async_copy(ones_v, acc.at[idx_d.at[b]],
                                 sem_s.at[b], add=True)

            return carry

        lax.fori_loop(0, n + 1, stages, 0)
        for b in range(nb):
            pltpu.make_async_copy(ones_v, acc.at[idx_d.at[b]],
                                  sem_s.at[b]).wait()
        plsc.subcore_barrier()
        pltpu.sync_copy(acc.at[pl.ds(s * ZROWS, ZROWS)],
                        out_hbm.at[pl.ds(c * NPAD + s * ZROWS, ZROWS)])

    return kdeg


NB = 3


def _make_edge_kernel(split_features):
    cores_per_edge = 1 if split_features else NC
    ep_tile = EP // (NS * cores_per_edge)
    n = ep_tile // K

    @functools.partial(
        pl.kernel,
        out_type=jax.ShapeDtypeStruct((NC * NPAD, W), jnp.float32),
        mesh=_mesh,
        scratch_types=[
            pltpu.VMEM_SHARED((NPAD, W), jnp.float32),
            pltpu.VMEM((NB, K), jnp.int32),
            pltpu.VMEM((NB, K), jnp.int32),
            pltpu.VMEM((NB, K, W), jnp.float32),
            pltpu.SemaphoreType.DMA((NB,)),
            pltpu.SemaphoreType.DMA((NB,)),
            pltpu.SemaphoreType.DMA((NB,)),
        ],
    )
    def kedge(tab_hbm, srca_hbm, srcb_hbm, dst_hbm, zin_hbm, out_hbm,
              acc, idx_s, idx_d, rows, sem_i, sem_g, sem_s):
        c = lax.axis_index("c")
        s = lax.axis_index("s")
        if split_features:
            tile_base = s * ep_tile
        else:
            tile_base = (s * NC + c) * ep_tile
        pltpu.sync_copy(zin_hbm, acc.at[pl.ds(s * ZROWS, ZROWS)])
        plsc.subcore_barrier()

        def stages(t, carry):
            @pl.when(t < n)
            def _a():
                b = lax.rem(t, NB)

                @pl.when(t >= NB)
                def _wait_free():
                    pltpu.make_async_copy(
                        rows.at[b], acc.at[idx_d.at[b]], sem_s.at[b]).wait()

                base = tile_base + t * K

                @pl.when(c == 0)
                def _lda():
                    pltpu.async_copy(srca_hbm.at[pl.ds(base, K)],
                                     idx_s.at[b], sem_i.at[b])

                @pl.when(c == 1)
                def _ldb():
                    pltpu.async_copy(srcb_hbm.at[pl.ds(base, K)],
                                     idx_s.at[b], sem_i.at[b])

                pltpu.async_copy(dst_hbm.at[pl.ds(base, K)],
                                 idx_d.at[b], sem_i.at[b])

            @pl.when(jnp.logical_and(t >= 1, t <= n))
            def _b():
                b = lax.rem(t - 1, NB)
                pltpu.make_async_copy(dst_hbm.at[pl.ds(0, K)],
                                      idx_s.at[b], sem_i.at[b]).wait()
                pltpu.make_async_copy(dst_hbm.at[pl.ds(0, K)],
                                      idx_d.at[b], sem_i.at[b]).wait()
                pltpu.async_copy(tab_hbm.at[idx_s.at[b]], rows.at[b],
                                 sem_g.at[b])

            @pl.when(t >= 2)
            def _c():
                b = lax.rem(t - 2, NB)
                pltpu.make_async_copy(tab_hbm.at[idx_s.at[b]], rows.at[b],
                                      sem_g.at[b]).wait()
                pltpu.async_copy(rows.at[b], acc.at[idx_d.at[b]],
                                 sem_s.at[b], add=True)

            return carry

        lax.fori_loop(0, n + 2, stages, 0)
        for b in range(NB):
            pltpu.make_async_copy(rows.at[b], acc.at[idx_d.at[b]],
                                  sem_s.at[b]).wait()
        plsc.subcore_barrier()
        pltpu.sync_copy(acc.at[pl.ds(s * ZROWS, ZROWS)],
                        out_hbm.at[pl.ds(c * NPAD + s * ZROWS, ZROWS)])

    return kedge


_deg_call = _make_deg_kernel()
_edge1_call = _make_edge_kernel(split_features=False)
_edge2_call = _make_edge_kernel(split_features=True)

_BLK = 1000
_NBLK = N // _BLK


def _scale_body(degp_ref, x_ref, xs_ref, dinv_ref):
    deg = 1.0 + degp_ref[0, :, 0] + degp_ref[1, :, 0]
    dinv = lax.rsqrt(deg)
    xs_ref[...] = dinv[:, None] * x_ref[...]
    dinv_ref[...] = dinv[:, None]


def _scale_call(degp3, x):
    return pl.pallas_call(
        _scale_body,
        grid=(_NBLK,),
        in_specs=[
            pl.BlockSpec((2, _BLK, WD), lambda i: (0, i, 0)),
            pl.BlockSpec((_BLK, 128), lambda i: (i, 0)),
        ],
        out_specs=[
            pl.BlockSpec((_BLK, 128), lambda i: (i, 0)),
            pl.BlockSpec((_BLK, 1), lambda i: (i, 0)),
        ],
        out_shape=[
            jax.ShapeDtypeStruct((N, 128), jnp.float32),
            jax.ShapeDtypeStruct((N, 1), jnp.float32),
        ],
    )(degp3, x)


def _layer1_body(u1_ref, xs_ref, dinv_ref, w1_ref, b1_ref, hs2_ref):
    dinv = dinv_ref[...]
    agg = (u1_ref[0] + u1_ref[1] + xs_ref[...]) * dinv
    h = jnp.dot(agg, w1_ref[...], preferred_element_type=jnp.float32)
    h = jnp.maximum(h + b1_ref[0], 0.0)
    hs = h * dinv
    hs2_ref[0] = hs[:, :128]
    hs2_ref[1] = hs[:, 128:]


def _layer1_call(u1p, xs, dinv, w1, b1):
    return pl.pallas_call(
        _layer1_body,
        grid=(_NBLK,),
        in_specs=[
            pl.BlockSpec((2, _BLK, W), lambda i: (0, i, 0)),
            pl.BlockSpec((_BLK, 128), lambda i: (i, 0)),
            pl.BlockSpec((_BLK, 1), lambda i: (i, 0)),
            pl.BlockSpec((128, 256), lambda i: (0, 0)),
            pl.BlockSpec((1, 256), lambda i: (0, 0)),
        ],
        out_specs=pl.BlockSpec((2, _BLK, 128), lambda i: (0, i, 0)),
        out_shape=jax.ShapeDtypeStruct((2, N, 128), jnp.float32),
    )(u1p, xs, dinv, w1, b1)


def _layer2_body(u2_ref, hs2_ref, dinv_ref, w2_ref, b2_ref, wfc_ref, bfc_ref,
                 batch_ref, out_ref, sums_s, cnts_s):
    i = pl.program_id(0)
    dinv = dinv_ref[...]
    t0 = u2_ref[0] + hs2_ref[0]
    t1 = u2_ref[1] + hs2_ref[1]
    agg = jnp.concatenate([t0, t1], axis=1) * dinv
    h = jnp.dot(agg, w2_ref[...], preferred_element_type=jnp.float32)
    h = jnp.maximum(h + b2_ref[0], 0.0)
    t = jnp.dot(h, wfc_ref[...], preferred_element_type=jnp.float32)[:, 0]
    oh = (batch_ref[...]
          == lax.broadcasted_iota(jnp.int32, (1, G), 1)).astype(jnp.float32)
    spart = jnp.dot(t, oh, preferred_element_type=jnp.float32)
    cpart = jnp.sum(oh, axis=0)

    @pl.when(i == 0)
    def _init():
        sums_s[...] = jnp.zeros_like(sums_s)
        cnts_s[...] = jnp.zeros_like(cnts_s)

    sums_s[0] = sums_s[0] + spart
    cnts_s[0] = cnts_s[0] + cpart

    @pl.when(i == _NBLK - 1)
    def _fin():
        out_ref[...] = (sums_s[0] / jnp.maximum(cnts_s[0], 1.0)
                        + bfc_ref[0])


def _layer2_call(u2, hs2, dinv, w2, b2, wfc, bfc, batch):
    return pl.pallas_call(
        _layer2_body,
        grid=(_NBLK,),
        in_specs=[
            pl.BlockSpec((2, _BLK, 128), lambda i: (0, i, 0)),
            pl.BlockSpec((2, _BLK, 128), lambda i: (0, i, 0)),
            pl.BlockSpec((_BLK, 1), lambda i: (i, 0)),
            pl.BlockSpec((256, 256), lambda i: (0, 0)),
            pl.BlockSpec((1, 256), lambda i: (0, 0)),
            pl.BlockSpec((256, 1), lambda i: (0, 0)),
            pl.BlockSpec((1,), lambda i: (0,)),
            pl.BlockSpec((_BLK, 1), lambda i: (i, 0)),
        ],
        out_specs=pl.BlockSpec((G,), lambda i: (0,)),
        out_shape=jax.ShapeDtypeStruct((G,), jnp.float32),
        scratch_shapes=[
            pltpu.VMEM((1, G), jnp.float32),
            pltpu.VMEM((1, G), jnp.float32),
        ],
    )(u2, hs2, dinv, w2, b2, wfc, bfc, batch)


def kernel(x, edge_index, batch, W1, b1, W2, b2, Wfc, bfc):
    npadd = EP - E
    src = jnp.concatenate([edge_index[0], jnp.zeros((npadd,), jnp.int32)])
    dst = jnp.concatenate([edge_index[1],
                           jnp.full((npadd,), DROW, jnp.int32)])
    zin = jnp.zeros((ZROWS, W), jnp.float32)
    zin_d = jnp.zeros((ZROWS, WD), jnp.float32)
    ones = jnp.zeros((K, WD), jnp.float32).at[:, 0].set(1.0)

    src_hi = src + N

    degp = _deg_call(dst, zin_d, ones)
    xs, dinv = _scale_call(degp.reshape(2, NPAD, WD), x)
    u1p = _edge1_call(xs, src, src, dst, zin)
    hs2 = _layer1_call(u1p.reshape(2, NPAD, W), xs, dinv, W1,
                       b1.reshape(1, -1))
    u2 = _edge2_call(hs2.reshape(2 * N, 128), src, src_hi, dst, zin)
    out = _layer2_call(u2.reshape(2, NPAD, 128), hs2, dinv, W2,
                       b2.reshape(1, -1), Wfc, bfc, batch.reshape(N, 1))
    return out


# speedup vs baseline: 1.8782x; 1.8782x over previous
import functools

import jax
import jax.numpy as jnp
from jax import lax
from jax.experimental import pallas as pl
from jax.experimental.pallas import tpu as pltpu
from jax.experimental.pallas import tpu_sc as plsc

N = 10000
E = 320000
G = 16
W = 128
NC = 2
NS = 16
NPAD = 10240
ZROWS = NPAD // NS
K = 96
EP = 322560
DROW = 10016

_mesh = plsc.VectorSubcoreMesh(core_axis_name="c", subcore_axis_name="s")


WD = 16


def _make_deg_kernel():
    ep_tile = EP // (NC * NS)
    n = ep_tile // K
    nb = 2

    @functools.partial(
        pl.kernel,
        out_type=jax.ShapeDtypeStruct((NC * NPAD, WD), jnp.float32),
        mesh=_mesh,
        scratch_types=[
            pltpu.VMEM_SHARED((NPAD, WD), jnp.float32),
            pltpu.VMEM((nb, K), jnp.int32),
            pltpu.VMEM((K, WD), jnp.float32),
            pltpu.SemaphoreType.DMA((nb,)),
            pltpu.SemaphoreType.DMA((nb,)),
        ],
    )
    def kdeg(dst_hbm, zin_hbm, ones_hbm, out_hbm, acc, idx_d, ones_v,
             sem_i, sem_s):
        c = lax.axis_index("c")
        s = lax.axis_index("s")
        wid = s * NC + c
        pltpu.sync_copy(zin_hbm, acc.at[pl.ds(s * ZROWS, ZROWS)])
        pltpu.sync_copy(ones_hbm, ones_v)
        plsc.subcore_barrier()

        def stages(t, carry):
            @pl.when(t < n)
            def _a():
                b = lax.rem(t, nb)

                @pl.when(t >= nb)
                def _wait_free():
                    pltpu.make_async_copy(
                        ones_v, acc.at[idx_d.at[b]], sem_s.at[b]).wait()

                base = wid * ep_tile + t * K
                pltpu.async_copy(dst_hbm.at[pl.ds(base, K)],
                                 idx_d.at[b], sem_i.at[b])

            @pl.when(t >= 1)
            def _c():
                b = lax.rem(t - 1, nb)
                pltpu.make_async_copy(dst_hbm.at[pl.ds(0, K)],
                                      idx_d.at[b], sem_i.at[b]).wait()
                pltpu.async_copy(ones_v, acc.at[idx_d.at[b]],
                                 sem_s.at[b], add=True)

            return carry

        lax.fori_loop(0, n + 1, stages, 0)
        for b in range(nb):
            pltpu.make_async_copy(ones_v, acc.at[idx_d.at[b]],
                                  sem_s.at[b]).wait()
        plsc.subcore_barrier()
        pltpu.sync_copy(acc.at[pl.ds(s * ZROWS, ZROWS)],
                        out_hbm.at[pl.ds(c * NPAD + s * ZROWS, ZROWS)])

    return kdeg


NB = 3


def _make_edge_kernel(split_features):
    cores_per_edge = 1 if split_features else NC
    ep_tile = EP // (NS * cores_per_edge)
    n = ep_tile // K

    @functools.partial(
        pl.kernel,
        out_type=jax.ShapeDtypeStruct((NC * NPAD, W), jnp.float32),
        mesh=_mesh,
        scratch_types=[
            pltpu.VMEM_SHARED((NPAD, W), jnp.float32),
            pltpu.VMEM((NB, K), jnp.int32),
            pltpu.VMEM((NB, K), jnp.int32),
            pltpu.VMEM((NB, K, W), jnp.float32),
            pltpu.SemaphoreType.DMA((NB,)),
            pltpu.SemaphoreType.DMA((NB,)),
            pltpu.SemaphoreType.DMA((NB,)),
        ],
    )
    def kedge(tab_hbm, srca_hbm, srcb_hbm, dst_hbm, zin_hbm, out_hbm,
              acc, idx_s, idx_d, rows, sem_i, sem_g, sem_s):
        c = lax.axis_index("c")
        s = lax.axis_index("s")
        if split_features:
            tile_base = s * ep_tile
        else:
            tile_base = (s * NC + c) * ep_tile
        pltpu.sync_copy(zin_hbm, acc.at[pl.ds(s * ZROWS, ZROWS)])
        plsc.subcore_barrier()

        def stages(t, carry):
            @pl.when(t < n)
            def _a():
                b = lax.rem(t, NB)

                @pl.when(t >= NB)
                def _wait_free():
                    pltpu.make_async_copy(
                        rows.at[b], acc.at[idx_d.at[b]], sem_s.at[b]).wait()

                base = tile_base + t * K

                @pl.when(c == 0)
                def _lda():
                    pltpu.async_copy(srca_hbm.at[pl.ds(base, K)],
                                     idx_s.at[b], sem_i.at[b])

                @pl.when(c == 1)
                def _ldb():
                    pltpu.async_copy(srcb_hbm.at[pl.ds(base, K)],
                                     idx_s.at[b], sem_i.at[b])

                pltpu.async_copy(dst_hbm.at[pl.ds(base, K)],
                                 idx_d.at[b], sem_i.at[b])

            @pl.when(jnp.logical_and(t >= 1, t <= n))
            def _b():
                b = lax.rem(t - 1, NB)
                pltpu.make_async_copy(dst_hbm.at[pl.ds(0, K)],
                                      idx_s.at[b], sem_i.at[b]).wait()
                pltpu.make_async_copy(dst_hbm.at[pl.ds(0, K)],
                                      idx_d.at[b], sem_i.at[b]).wait()
                pltpu.async_copy(tab_hbm.at[idx_s.at[b]], rows.at[b],
                                 sem_g.at[b])

            @pl.when(t >= 2)
            def _c():
                b = lax.rem(t - 2, NB)
                pltpu.make_async_copy(tab_hbm.at[idx_s.at[b]], rows.at[b],
                                      sem_g.at[b]).wait()
                pltpu.async_copy(rows.at[b], acc.at[idx_d.at[b]],
                                 sem_s.at[b], add=True)

            return carry

        lax.fori_loop(0, n + 2, stages, 0)
        for b in range(NB):
            pltpu.make_async_copy(rows.at[b], acc.at[idx_d.at[b]],
                                  sem_s.at[b]).wait()
        plsc.subcore_barrier()
        pltpu.sync_copy(acc.at[pl.ds(s * ZROWS, ZROWS)],
                        out_hbm.at[pl.ds(c * NPAD + s * ZROWS, ZROWS)])

    return kedge


_deg_call = _make_deg_kernel()
_edge1_call = _make_edge_kernel(split_features=False)
_edge2_call = _make_edge_kernel(split_features=True)

_BLK = 1000
_NBLK = N // _BLK


def _scale_body(degp_ref, x_ref, xs_ref, dinv_ref):
    deg = 1.0 + degp_ref[0, :, 0] + degp_ref[1, :, 0]
    dinv = lax.rsqrt(deg)
    xs_ref[...] = dinv[:, None] * x_ref[...]
    dinv_ref[...] = dinv[:, None]


def _scale_call(degp3, x):
    return pl.pallas_call(
        _scale_body,
        grid=(_NBLK,),
        in_specs=[
            pl.BlockSpec((2, _BLK, WD), lambda i: (0, i, 0)),
            pl.BlockSpec((_BLK, 128), lambda i: (i, 0)),
        ],
        out_specs=[
            pl.BlockSpec((_BLK, 128), lambda i: (i, 0)),
            pl.BlockSpec((_BLK, 1), lambda i: (i, 0)),
        ],
        out_shape=[
            jax.ShapeDtypeStruct((N, 128), jnp.float32),
            jax.ShapeDtypeStruct((N, 1), jnp.float32),
        ],
    )(degp3, x)


def _layer1_body(u1_ref, xs_ref, dinv_ref, w1_ref, b1_ref, hs2_ref):
    dinv = dinv_ref[...]
    agg = (u1_ref[0] + u1_ref[1] + xs_ref[...]) * dinv
    h = jnp.dot(agg, w1_ref[...], preferred_element_type=jnp.float32)
    h = jnp.maximum(h + b1_ref[0], 0.0)
    hs = h * dinv
    hs2_ref[0] = hs[:, :128]
    hs2_ref[1] = hs[:, 128:]


def _layer1_call(u1p, xs, dinv, w1, b1):
    return pl.pallas_call(
        _layer1_body,
        grid=(_NBLK,),
        in_specs=[
            pl.BlockSpec((2, _BLK, W), lambda i: (0, i, 0)),
            pl.BlockSpec((_BLK, 128), lambda i: (i, 0)),
            pl.BlockSpec((_BLK, 1), lambda i: (i, 0)),
            pl.BlockSpec((128, 256), lambda i: (0, 0)),
            pl.BlockSpec((1, 256), lambda i: (0, 0)),
        ],
        out_specs=pl.BlockSpec((2, _BLK, 128), lambda i: (0, i, 0)),
        out_shape=jax.ShapeDtypeStruct((2, N, 128), jnp.float32),
    )(u1p, xs, dinv, w1, b1)


def _layer2_body(u2_ref, hs2_ref, dinv_ref, w2_ref, b2_ref, wfc_ref, bfc_ref,
                 batch_ref, out_ref, sums_s, cnts_s):
    i = pl.program_id(0)
    dinv = dinv_ref[...]
    t0 = u2_ref[0] + hs2_ref[0]
    t1 = u2_ref[1] + hs2_ref[1]
    agg = jnp.concatenate([t0, t1], axis=1) * dinv
    h = jnp.dot(agg, w2_ref[...], preferred_element_type=jnp.float32)
    h = jnp.maximum(h + b2_ref[0], 0.0)
    t = jnp.dot(h, wfc_ref[...], preferred_element_type=jnp.float32)[:, 0]
    oh = (batch_ref[...]
          == lax.broadcasted_iota(jnp.int32, (1, G), 1)).astype(jnp.float32)
    spart = jnp.dot(t, oh, preferred_element_type=jnp.float32)
    cpart = jnp.sum(oh, axis=0)

    @pl.when(i == 0)
    def _init():
        sums_s[...] = jnp.zeros_like(sums_s)
        cnts_s[...] = jnp.zeros_like(cnts_s)

    sums_s[0] = sums_s[0] + spart
    cnts_s[0] = cnts_s[0] + cpart

    @pl.when(i == _NBLK - 1)
    def _fin():
        out_ref[...] = (sums_s[0] / jnp.maximum(cnts_s[0], 1.0)
                        + bfc_ref[0])


def _layer2_call(u2, hs2, dinv, w2, b2, wfc, bfc, batch):
    return pl.pallas_call(
        _layer2_body,
        grid=(_NBLK,),
        in_specs=[
            pl.BlockSpec((2, _BLK, 128), lambda i: (0, i, 0)),
            pl.BlockSpec((2, _BLK, 128), lambda i: (0, i, 0)),
            pl.BlockSpec((_BLK, 1), lambda i: (i, 0)),
            pl.BlockSpec((256, 256), lambda i: (0, 0)),
            pl.BlockSpec((1, 256), lambda i: (0, 0)),
            pl.BlockSpec((256, 1), lambda i: (0, 0)),
            pl.BlockSpec((1,), lambda i: (0,)),
            pl.BlockSpec((_BLK, 1), lambda i: (i, 0)),
        ],
        out_specs=pl.BlockSpec((G,), lambda i: (0,)),
        out_shape=jax.ShapeDtypeStruct((G,), jnp.float32),
        scratch_shapes=[
            pltpu.VMEM((1, G), jnp.float32),
            pltpu.VMEM((1, G), jnp.float32),
        ],
    )(u2, hs2, dinv, w2, b2, wfc, bfc, batch)


def kernel(x, edge_index, batch, W1, b1, W2, b2, Wfc, bfc):
    npadd = EP - E
    src = jnp.concatenate([edge_index[0], jnp.zeros((npadd,), jnp.int32)])
    dst = jnp.concatenate([edge_index[1],
                           jnp.full((npadd,), DROW, jnp.int32)])
    zin = jnp.zeros((ZROWS, W), jnp.float32)
    zin_d = jnp.zeros((ZROWS, WD), jnp.float32)
    ones = jnp.zeros((K, WD), jnp.float32).at[:, 0].set(1.0)

    src_hi = src + N

    degp = _deg_call(dst, zin_d, ones)
    xs, dinv = _scale_call(degp.reshape(2, NPAD, WD), x)
    u1p = _edge1_call(xs, src, src, dst, zin)
    hs2 = _layer1_call(u1p.reshape(2, NPAD, W), xs, dinv, W1,
                       b1.reshape(1, -1))
    u2 = _edge2_call(hs2.reshape(2 * N, 128), src, src_hi, dst, zin)
    out = _layer2_call(u2.reshape(2, NPAD, 128), hs2, dinv, W2,
                       b2.reshape(1, -1), Wfc, bfc, batch.reshape(N, 1))
    return out


# speedup vs baseline: 2.9825x; 1.5880x over previous
import functools

import jax
import jax.numpy as jnp
from jax import lax
from jax.experimental import pallas as pl
from jax.experimental.pallas import tpu as pltpu
from jax.experimental.pallas import tpu_sc as plsc

N = 10000
E = 320000
G = 16
W = 128
NC = 2
NS = 16
NPAD = 10240
ZROWS = NPAD // NS
K = 80
EP = 320000
DROW = 10016

_mesh = plsc.VectorSubcoreMesh(core_axis_name="c", subcore_axis_name="s")


WD = 16


def _make_deg_kernel():
    ep_tile = EP // (NC * NS)
    n = ep_tile // K
    nb = 2

    @functools.partial(
        pl.kernel,
        out_type=jax.ShapeDtypeStruct((NC * NPAD, WD), jnp.float32),
        mesh=_mesh,
        scratch_types=[
            pltpu.VMEM_SHARED((NPAD, WD), jnp.float32),
            pltpu.VMEM((nb, K), jnp.int32),
            pltpu.VMEM((K, WD), jnp.float32),
            pltpu.SemaphoreType.DMA((nb,)),
            pltpu.SemaphoreType.DMA((nb,)),
        ],
    )
    def kdeg(dst_hbm, zin_hbm, ones_hbm, out_hbm, acc, idx_d, ones_v,
             sem_i, sem_s):
        c = lax.axis_index("c")
        s = lax.axis_index("s")
        wid = s * NC + c
        pltpu.sync_copy(zin_hbm, acc.at[pl.ds(s * ZROWS, ZROWS)])
        pltpu.sync_copy(ones_hbm, ones_v)
        plsc.subcore_barrier()

        def stages(t, carry):
            @pl.when(t < n)
            def _a():
                b = lax.rem(t, nb)

                @pl.when(t >= nb)
                def _wait_free():
                    pltpu.make_async_copy(
                        ones_v, acc.at[idx_d.at[b]], sem_s.at[b]).wait()

                base = wid * ep_tile + t * K
                pltpu.async_copy(dst_hbm.at[pl.ds(base, K)],
                                 idx_d.at[b], sem_i.at[b])

            @pl.when(t >= 1)
            def _c():
                b = lax.rem(t - 1, nb)
                pltpu.make_async_copy(dst_hbm.at[pl.ds(0, K)],
                                      idx_d.at[b], sem_i.at[b]).wait()
                pltpu.async_copy(ones_v, acc.at[idx_d.at[b]],
                                 sem_s.at[b], add=True)

            return carry

        lax.fori_loop(0, n + 1, stages, 0)
        for b in range(nb):
            pltpu.make_async_copy(ones_v, acc.at[idx_d.at[b]],
                                  sem_s.at[b]).wait()
        plsc.subcore_barrier()
        pltpu.sync_copy(acc.at[pl.ds(s * ZROWS, ZROWS)],
                        out_hbm.at[pl.ds(c * NPAD + s * ZROWS, ZROWS)])

    return kdeg


NB = 4


def _make_edge_kernel(split_features):
    cores_per_edge = 1 if split_features else NC
    ep_tile = EP // (NS * cores_per_edge)
    n = ep_tile // K

    @functools.partial(
        pl.kernel,
        out_type=jax.ShapeDtypeStruct((NC * NPAD, W), jnp.float32),
        mesh=_mesh,
        scratch_types=[
            pltpu.VMEM_SHARED((NPAD, W), jnp.float32),
            pltpu.VMEM((NB, K), jnp.int32),
            pltpu.VMEM((NB, K), jnp.int32),
            pltpu.VMEM((NB, K, W), jnp.float32),
            pltpu.SemaphoreType.DMA((NB,)),
            pltpu.SemaphoreType.DMA((NB,)),
            pltpu.SemaphoreType.DMA((NB,)),
        ],
    )
    def kedge(tab_hbm, srca_hbm, srcb_hbm, dst_hbm, zin_hbm, out_hbm,
              acc, idx_s, idx_d, rows, sem_i, sem_g, sem_s):
        c = lax.axis_index("c")
        s = lax.axis_index("s")
        if split_features:
            tile_base = s * ep_tile
        else:
            tile_base = (s * NC + c) * ep_tile
        pltpu.sync_copy(zin_hbm, acc.at[pl.ds(s * ZROWS, ZROWS)])
        plsc.subcore_barrier()

        def stages(t, carry):
            @pl.when(t < n)
            def _a():
                b = lax.rem(t, NB)

                @pl.when(t >= NB)
                def _wait_free():
                    pltpu.make_async_copy(
                        rows.at[b], acc.at[idx_d.at[b]], sem_s.at[b]).wait()

                base = tile_base + t * K

                @pl.when(c == 0)
                def _lda():
                    pltpu.async_copy(srca_hbm.at[pl.ds(base, K)],
                                     idx_s.at[b], sem_i.at[b])

                @pl.when(c == 1)
                def _ldb():
                    pltpu.async_copy(srcb_hbm.at[pl.ds(base, K)],
                                     idx_s.at[b], sem_i.at[b])

                pltpu.async_copy(dst_hbm.at[pl.ds(base, K)],
                                 idx_d.at[b], sem_i.at[b])

            @pl.when(jnp.logical_and(t >= 1, t <= n))
            def _b():
                b = lax.rem(t - 1, NB)
                pltpu.make_async_copy(dst_hbm.at[pl.ds(0, K)],
                                      idx_s.at[b], sem_i.at[b]).wait()
                pltpu.make_async_copy(dst_hbm.at[pl.ds(0, K)],
                                      idx_d.at[b], sem_i.at[b]).wait()
                pltpu.async_copy(tab_hbm.at[idx_s.at[b]], rows.at[b],
                                 sem_g.at[b])

            @pl.when(t >= 2)
            def _c():
                b = lax.rem(t - 2, NB)
                pltpu.make_async_copy(tab_hbm.at[idx_s.at[b]], rows.at[b],
                                      sem_g.at[b]).wait()
                pltpu.async_copy(rows.at[b], acc.at[idx_d.at[b]],
                                 sem_s.at[b], add=True)

            return carry

        lax.fori_loop(0, n + 2, stages, 0)
        for b in range(NB):
            pltpu.make_async_copy(rows.at[b], acc.at[idx_d.at[b]],
                                  sem_s.at[b]).wait()
        plsc.subcore_barrier()
        pltpu.sync_copy(acc.at[pl.ds(s * ZROWS, ZROWS)],
                        out_hbm.at[pl.ds(c * NPAD + s * ZROWS, ZROWS)])

    return kedge


_deg_call = _make_deg_kernel()
_edge1_call = _make_edge_kernel(split_features=False)
_edge2_call = _make_edge_kernel(split_features=True)

_BLK = 1000
_NBLK = N // _BLK


def _scale_body(degp_ref, x_ref, xs_ref, dinv_ref):
    deg = 1.0 + degp_ref[0, :, 0] + degp_ref[1, :, 0]
    dinv = lax.rsqrt(deg)
    xs_ref[...] = dinv[:, None] * x_ref[...]
    dinv_ref[...] = dinv[:, None]


def _scale_call(degp3, x):
    return pl.pallas_call(
        _scale_body,
        grid=(_NBLK,),
        in_specs=[
            pl.BlockSpec((2, _BLK, WD), lambda i: (0, i, 0)),
            pl.BlockSpec((_BLK, 128), lambda i: (i, 0)),
        ],
        out_specs=[
            pl.BlockSpec((_BLK, 128), lambda i: (i, 0)),
            pl.BlockSpec((_BLK, 1), lambda i: (i, 0)),
        ],
        out_shape=[
            jax.ShapeDtypeStruct((N, 128), jnp.float32),
            jax.ShapeDtypeStruct((N, 1), jnp.float32),
        ],
    )(degp3, x)


def _layer1_body(u1_ref, xs_ref, dinv_ref, w1_ref, b1_ref, hs2_ref):
    dinv = dinv_ref[...]
    agg = (u1_ref[0] + u1_ref[1] + xs_ref[...]) * dinv
    h = jnp.dot(agg, w1_ref[...], preferred_element_type=jnp.float32)
    h = jnp.maximum(h + b1_ref[0], 0.0)
    hs = h * dinv
    hs2_ref[0] = hs[:, :128]
    hs2_ref[1] = hs[:, 128:]


def _layer1_call(u1p, xs, dinv, w1, b1):
    return pl.pallas_call(
        _layer1_body,
        grid=(_NBLK,),
        in_specs=[
            pl.BlockSpec((2, _BLK, W), lambda i: (0, i, 0)),
            pl.BlockSpec((_BLK, 128), lambda i: (i, 0)),
            pl.BlockSpec((_BLK, 1), lambda i: (i, 0)),
            pl.BlockSpec((128, 256), lambda i: (0, 0)),
            pl.BlockSpec((1, 256), lambda i: (0, 0)),
        ],
        out_specs=pl.BlockSpec((2, _BLK, 128), lambda i: (0, i, 0)),
        out_shape=jax.ShapeDtypeStruct((2, N, 128), jnp.float32),
    )(u1p, xs, dinv, w1, b1)


def _layer2_body(u2_ref, hs2_ref, dinv_ref, w2_ref, b2_ref, wfc_ref, bfc_ref,
                 batch_ref, out_ref, sums_s, cnts_s):
    i = pl.program_id(0)
    dinv = dinv_ref[...]
    t0 = u2_ref[0] + hs2_ref[0]
    t1 = u2_ref[1] + hs2_ref[1]
    agg = jnp.concatenate([t0, t1], axis=1) * dinv
    h = jnp.dot(agg, w2_ref[...], preferred_element_type=jnp.float32)
    h = jnp.maximum(h + b2_ref[0], 0.0)
    t = jnp.dot(h, wfc_ref[...], preferred_element_type=jnp.float32)[:, 0]
    oh = (batch_ref[...]
          == lax.broadcasted_iota(jnp.int32, (1, G), 1)).astype(jnp.float32)
    spart = jnp.dot(t, oh, preferred_element_type=jnp.float32)
    cpart = jnp.sum(oh, axis=0)

    @pl.when(i == 0)
    def _init():
        sums_s[...] = jnp.zeros_like(sums_s)
        cnts_s[...] = jnp.zeros_like(cnts_s)

    sums_s[0] = sums_s[0] + spart
    cnts_s[0] = cnts_s[0] + cpart

    @pl.when(i == _NBLK - 1)
    def _fin():
        out_ref[...] = (sums_s[0] / jnp.maximum(cnts_s[0], 1.0)
                        + bfc_ref[0])


def _layer2_call(u2, hs2, dinv, w2, b2, wfc, bfc, batch):
    return pl.pallas_call(
        _layer2_body,
        grid=(_NBLK,),
        in_specs=[
            pl.BlockSpec((2, _BLK, 128), lambda i: (0, i, 0)),
            pl.BlockSpec((2, _BLK, 128), lambda i: (0, i, 0)),
            pl.BlockSpec((_BLK, 1), lambda i: (i, 0)),
            pl.BlockSpec((256, 256), lambda i: (0, 0)),
            pl.BlockSpec((1, 256), lambda i: (0, 0)),
            pl.BlockSpec((256, 1), lambda i: (0, 0)),
            pl.BlockSpec((1,), lambda i: (0,)),
            pl.BlockSpec((_BLK, 1), lambda i: (i, 0)),
        ],
        out_specs=pl.BlockSpec((G,), lambda i: (0,)),
        out_shape=jax.ShapeDtypeStruct((G,), jnp.float32),
        scratch_shapes=[
            pltpu.VMEM((1, G), jnp.float32),
            pltpu.VMEM((1, G), jnp.float32),
        ],
    )(u2, hs2, dinv, w2, b2, wfc, bfc, batch)


def kernel(x, edge_index, batch, W1, b1, W2, b2, Wfc, bfc):
    npadd = EP - E
    src = jnp.concatenate([edge_index[0], jnp.zeros((npadd,), jnp.int32)])
    dst = jnp.concatenate([edge_index[1],
                           jnp.full((npadd,), DROW, jnp.int32)])
    zin = jnp.zeros((ZROWS, W), jnp.float32)
    zin_d = jnp.zeros((ZROWS, WD), jnp.float32)
    ones = jnp.zeros((K, WD), jnp.float32).at[:, 0].set(1.0)

    src_hi = src + N

    degp = _deg_call(dst, zin_d, ones)
    xs, dinv = _scale_call(degp.reshape(2, NPAD, WD), x)
    u1p = _edge1_call(xs, src, src, dst, zin)
    hs2 = _layer1_call(u1p.reshape(2, NPAD, W), xs, dinv, W1,
                       b1.reshape(1, -1))
    u2 = _edge2_call(hs2.reshape(2 * N, 128), src, src_hi, dst, zin)
    out = _layer2_call(u2.reshape(2, NPAD, 128), hs2, dinv, W2,
                       b2.reshape(1, -1), Wfc, bfc, batch.reshape(N, 1))
    return out
